# Initial kernel scaffold; baseline (speedup 1.0000x reference)
import jax
import jax.numpy as jnp
from jax.experimental import pallas as pl


def kernel(x, A_hat, temp):
    raise NotImplementedError("write your pallas kernel here")


# speedup vs baseline: 1.2384x; 1.2384x over previous
import jax
import jax.numpy as jnp
from jax.experimental import pallas as pl
from jax.experimental.pallas import tpu as pltpu

_K = 10
_N = 10000
_D = 128
_NP = 10240
_RT = 512
_T = _NP // _RT


def _gpr_body(temp_ref, x_ref, a_ref, out_ref, acc_ref, h_ref):
    k = pl.program_id(0)
    t = pl.program_id(1)

    @pl.when((k == 0) & (t == 0))
    def _init():
        h_ref[0] = x_ref[...]

    rd = jax.lax.rem(k, 2)
    wr = 1 - rd

    a = a_ref[...]
    h_old = h_ref[rd]
    h_new = jnp.dot(a, h_old, preferred_element_type=jnp.float32)

    row0 = t * _RT
    h_ref[wr, pl.ds(row0, _RT), :] = h_new.astype(jnp.bfloat16)

    tk = temp_ref[k + 1]

    @pl.when(k == 0)
    def _acc0():
        x_rows = x_ref[pl.ds(row0, _RT), :].astype(jnp.float32)
        acc_ref[pl.ds(row0, _RT), :] = temp_ref[0] * x_rows + tk * h_new

    @pl.when(k > 0)
    def _acck():
        acc_ref[pl.ds(row0, _RT), :] = (
            acc_ref[pl.ds(row0, _RT), :] + tk * h_new
        )

    @pl.when(k == _K - 1)
    def _emit():
        out_ref[...] = acc_ref[pl.ds(row0, _RT), :]


def kernel(x, A_hat, temp):
    a_b = jnp.pad(A_hat.astype(jnp.bfloat16),
                  ((0, _NP - _N), (0, _NP - _N)))
    x_b = jnp.pad(x.astype(jnp.bfloat16), ((0, _NP - _N), (0, 0)))

    grid_spec = pltpu.PrefetchScalarGridSpec(
        num_scalar_prefetch=1,
        grid=(_K, _T),
        in_specs=[
            pl.BlockSpec((_NP, _D), lambda k, t, *_: (0, 0)),
            pl.BlockSpec((_RT, _NP), lambda k, t, *_: (t, 0)),
        ],
        out_specs=pl.BlockSpec(
            (_RT, _D), lambda k, t, *_: (jnp.where(k == _K - 1, t, 0), 0)),
        scratch_shapes=[
            pltpu.VMEM((_NP, _D), jnp.float32),
            pltpu.VMEM((2, _NP, _D), jnp.bfloat16),
        ],
    )

    out = pl.pallas_call(
        _gpr_body,
        grid_spec=grid_spec,
        out_shape=jax.ShapeDtypeStruct((_NP, _D), jnp.float32),
        compiler_params=pltpu.CompilerParams(
            dimension_semantics=("arbitrary", "arbitrary"),
        ),
    )(temp, x_b, a_b)

    return out[:_N]


# speedup vs baseline: 1.5536x; 1.2545x over previous
import jax
import jax.numpy as jnp
from jax.experimental import pallas as pl
from jax.experimental.pallas import tpu as pltpu

_K = 10
_N = 10000
_D = 128
_NS = 10240
_RT1 = 400
_T1 = _N // _RT1
_RT2 = 512
_T2 = -(-_N // _RT2)


def _hop0_body(temp_ref, x_ref, a_ref, ab_ref, h1_ref, acc1_ref):
    t = pl.program_id(0)
    a_bf = a_ref[...].astype(jnp.bfloat16)
    ab_ref[...] = a_bf
    h_new = jnp.dot(a_bf, x_ref[...],
                    preferred_element_type=jnp.float32)
    h1_ref[...] = h_new.astype(jnp.bfloat16)
    x_rows = x_ref[pl.ds(t * _RT1, _RT1), :].astype(jnp.float32)
    acc1_ref[...] = temp_ref[0] * x_rows + temp_ref[1] * h_new


def _hops_body(temp_ref, h1_ref, acc1_ref, ab_ref, out_ref, acc_ref, h_ref):
    k = pl.program_id(0)
    t = pl.program_id(1)

    @pl.when((k == 0) & (t == 0))
    def _init():
        h_ref[0, pl.ds(0, _N), :] = h1_ref[...]
        acc_ref[pl.ds(0, _N), :] = acc1_ref[...]

    rd = jax.lax.rem(k, 2)
    wr = 1 - rd

    a = ab_ref[...]
    h_old = h_ref[rd, pl.ds(0, _N), :]
    h_new = jnp.dot(a, h_old, preferred_element_type=jnp.float32)

    row0 = t * _RT2
    h_ref[wr, pl.ds(row0, _RT2), :] = h_new.astype(jnp.bfloat16)
    acc_ref[pl.ds(row0, _RT2), :] = (
        acc_ref[pl.ds(row0, _RT2), :] + temp_ref[k + 2] * h_new
    )

    @pl.when(k == _K - 2)
    def _emit():
        out_ref[...] = acc_ref[pl.ds(row0, _RT2), :]


def kernel(x, A_hat, temp):
    x_b = x.astype(jnp.bfloat16)

    hop0 = pltpu.PrefetchScalarGridSpec(
        num_scalar_prefetch=1,
        grid=(_T1,),
        in_specs=[
            pl.BlockSpec((_N, _D), lambda t, *_: (0, 0)),
            pl.BlockSpec((_RT1, _N), lambda t, *_: (t, 0)),
        ],
        out_specs=[
            pl.BlockSpec((_RT1, _N), lambda t, *_: (t, 0)),
            pl.BlockSpec((_RT1, _D), lambda t, *_: (t, 0)),
            pl.BlockSpec((_RT1, _D), lambda t, *_: (t, 0)),
        ],
    )
    a_b, h1, acc1 = pl.pallas_call(
        _hop0_body,
        grid_spec=hop0,
        out_shape=[
            jax.ShapeDtypeStruct((_N, _N), jnp.bfloat16),
            jax.ShapeDtypeStruct((_N, _D), jnp.bfloat16),
            jax.ShapeDtypeStruct((_N, _D), jnp.float32),
        ],
        compiler_params=pltpu.CompilerParams(
            dimension_semantics=("arbitrary",),
        ),
    )(temp, x_b, A_hat)

    hops = pltpu.PrefetchScalarGridSpec(
        num_scalar_prefetch=1,
        grid=(_K - 1, _T2),
        in_specs=[
            pl.BlockSpec((_N, _D), lambda k, t, *_: (0, 0)),
            pl.BlockSpec((_N, _D), lambda k, t, *_: (0, 0)),
            pl.BlockSpec((_RT2, _N), lambda k, t, *_: (t, 0)),
        ],
        out_specs=pl.BlockSpec(
            (_RT2, _D), lambda k, t, *_: (jnp.where(k == _K - 2, t, 0), 0)),
        scratch_shapes=[
            pltpu.VMEM((_NS, _D), jnp.float32),
            pltpu.VMEM((2, _NS, _D), jnp.bfloat16),
        ],
    )
    out = pl.pallas_call(
        _hops_body,
        grid_spec=hops,
        out_shape=jax.ShapeDtypeStruct((_N, _D), jnp.float32),
        compiler_params=pltpu.CompilerParams(
            dimension_semantics=("arbitrary", "arbitrary"),
        ),
    )(temp, h1, acc1, a_b)

    return out


# speedup vs baseline: 1.5969x; 1.0279x over previous
import jax
import jax.numpy as jnp
from jax.experimental import pallas as pl
from jax.experimental.pallas import tpu as pltpu

_K = 10
_N = 10000
_D = 128
_NS = 10240
_RT1 = 400
_T1 = _N // _RT1
_RT2 = 640
_T2 = -(-_N // _RT2)


def _hop0_body(temp_ref, x_ref, a_ref, ab_ref, h1_ref, acc1_ref):
    t = pl.program_id(0)
    a_bf = a_ref[...].astype(jnp.bfloat16)
    ab_ref[...] = a_bf
    h_new = jnp.dot(a_bf, x_ref[...],
                    preferred_element_type=jnp.float32)
    h1_ref[...] = h_new.astype(jnp.bfloat16)
    x_rows = x_ref[pl.ds(t * _RT1, _RT1), :].astype(jnp.float32)
    acc1_ref[...] = temp_ref[0] * x_rows + temp_ref[1] * h_new


def _hops_body(temp_ref, h1_ref, acc1_ref, ab_ref, out_ref, acc_ref, h_ref):
    k = pl.program_id(0)
    t = pl.program_id(1)

    @pl.when((k == 0) & (t == 0))
    def _init():
        h_ref[0, pl.ds(0, _N), :] = h1_ref[...]
        acc_ref[pl.ds(0, _N), :] = acc1_ref[...]

    rd = jax.lax.rem(k, 2)
    wr = 1 - rd

    a = ab_ref[...]
    h_old = h_ref[rd, pl.ds(0, _N), :]
    h_new = jnp.dot(a, h_old, preferred_element_type=jnp.float32)

    row0 = t * _RT2
    h_ref[wr, pl.ds(row0, _RT2), :] = h_new.astype(jnp.bfloat16)
    acc_ref[pl.ds(row0, _RT2), :] = (
        acc_ref[pl.ds(row0, _RT2), :] + temp_ref[k + 2] * h_new
    )

    @pl.when(k == _K - 2)
    def _emit():
        out_ref[...] = acc_ref[pl.ds(row0, _RT2), :]


def kernel(x, A_hat, temp):
    x_b = x.astype(jnp.bfloat16)

    hop0 = pltpu.PrefetchScalarGridSpec(
        num_scalar_prefetch=1,
        grid=(_T1,),
        in_specs=[
            pl.BlockSpec((_N, _D), lambda t, *_: (0, 0)),
            pl.BlockSpec((_RT1, _N), lambda t, *_: (t, 0)),
        ],
        out_specs=[
            pl.BlockSpec((_RT1, _N), lambda t, *_: (t, 0)),
            pl.BlockSpec((_RT1, _D), lambda t, *_: (t, 0)),
            pl.BlockSpec((_RT1, _D), lambda t, *_: (t, 0)),
        ],
    )
    a_b, h1, acc1 = pl.pallas_call(
        _hop0_body,
        grid_spec=hop0,
        out_shape=[
            jax.ShapeDtypeStruct((_N, _N), jnp.bfloat16),
            jax.ShapeDtypeStruct((_N, _D), jnp.bfloat16),
            jax.ShapeDtypeStruct((_N, _D), jnp.float32),
        ],
        compiler_params=pltpu.CompilerParams(
            dimension_semantics=("arbitrary",),
        ),
    )(temp, x_b, A_hat)

    hops = pltpu.PrefetchScalarGridSpec(
        num_scalar_prefetch=1,
        grid=(_K - 1, _T2),
        in_specs=[
            pl.BlockSpec((_N, _D), lambda k, t, *_: (0, 0)),
            pl.BlockSpec((_N, _D), lambda k, t, *_: (0, 0)),
            pl.BlockSpec((_RT2, _N), lambda k, t, *_: (t, 0)),
        ],
        out_specs=pl.BlockSpec(
            (_RT2, _D), lambda k, t, *_: (jnp.where(k == _K - 2, t, 0), 0)),
        scratch_shapes=[
            pltpu.VMEM((_NS, _D), jnp.float32),
            pltpu.VMEM((2, _NS, _D), jnp.bfloat16),
        ],
    )
    out = pl.pallas_call(
        _hops_body,
        grid_spec=hops,
        out_shape=jax.ShapeDtypeStruct((_N, _D), jnp.float32),
        compiler_params=pltpu.CompilerParams(
            dimension_semantics=("arbitrary", "arbitrary"),
        ),
    )(temp, h1, acc1, a_b)

    return out


# speedup vs baseline: 1.6367x; 1.0249x over previous
import jax
import jax.numpy as jnp
from jax.experimental import pallas as pl
from jax.experimental.pallas import tpu as pltpu

_K = 10
_N = 10000
_D = 128
_NS = 10400
_RT1 = 400
_T1 = _N // _RT1
_RT2 = 800
_T2 = -(-_N // _RT2)


def _hop0_body(temp_ref, x_ref, a_ref, ab_ref, h1_ref, acc1_ref):
    t = pl.program_id(0)
    a_bf = a_ref[...].astype(jnp.bfloat16)
    ab_ref[...] = a_bf
    h_new = jnp.dot(a_bf, x_ref[...],
                    preferred_element_type=jnp.float32)
    h1_ref[...] = h_new.astype(jnp.bfloat16)
    x_rows = x_ref[pl.ds(t * _RT1, _RT1), :].astype(jnp.float32)
    acc1_ref[...] = temp_ref[0] * x_rows + temp_ref[1] * h_new


def _hops_body(temp_ref, h1_ref, acc1_ref, ab_ref, out_ref, acc_ref, h_ref):
    k = pl.program_id(0)
    t = pl.program_id(1)

    @pl.when((k == 0) & (t == 0))
    def _init():
        h_ref[0, pl.ds(0, _N), :] = h1_ref[...]
        acc_ref[pl.ds(0, _N), :] = acc1_ref[...]

    rd = jax.lax.rem(k, 2)
    wr = 1 - rd

    a = ab_ref[...]
    h_old = h_ref[rd, pl.ds(0, _N), :]
    h_new = jnp.dot(a, h_old, preferred_element_type=jnp.float32)

    row0 = t * _RT2
    h_ref[wr, pl.ds(row0, _RT2), :] = h_new.astype(jnp.bfloat16)
    acc_ref[pl.ds(row0, _RT2), :] = (
        acc_ref[pl.ds(row0, _RT2), :] + temp_ref[k + 2] * h_new
    )

    @pl.when(k == _K - 2)
    def _emit():
        out_ref[...] = acc_ref[pl.ds(row0, _RT2), :]


def kernel(x, A_hat, temp):
    x_b = x.astype(jnp.bfloat16)

    hop0 = pltpu.PrefetchScalarGridSpec(
        num_scalar_prefetch=1,
        grid=(_T1,),
        in_specs=[
            pl.BlockSpec((_N, _D), lambda t, *_: (0, 0)),
            pl.BlockSpec((_RT1, _N), lambda t, *_: (t, 0)),
        ],
        out_specs=[
            pl.BlockSpec((_RT1, _N), lambda t, *_: (t, 0)),
            pl.BlockSpec((_RT1, _D), lambda t, *_: (t, 0)),
            pl.BlockSpec((_RT1, _D), lambda t, *_: (t, 0)),
        ],
    )
    a_b, h1, acc1 = pl.pallas_call(
        _hop0_body,
        grid_spec=hop0,
        out_shape=[
            jax.ShapeDtypeStruct((_N, _N), jnp.bfloat16),
            jax.ShapeDtypeStruct((_N, _D), jnp.bfloat16),
            jax.ShapeDtypeStruct((_N, _D), jnp.float32),
        ],
        compiler_params=pltpu.CompilerParams(
            dimension_semantics=("arbitrary",),
        ),
    )(temp, x_b, A_hat)

    hops = pltpu.PrefetchScalarGridSpec(
        num_scalar_prefetch=1,
        grid=(_K - 1, _T2),
        in_specs=[
            pl.BlockSpec((_N, _D), lambda k, t, *_: (0, 0)),
            pl.BlockSpec((_N, _D), lambda k, t, *_: (0, 0)),
            pl.BlockSpec((_RT2, _N), lambda k, t, *_: (t, 0)),
        ],
        out_specs=pl.BlockSpec(
            (_RT2, _D), lambda k, t, *_: (jnp.where(k == _K - 2, t, 0), 0)),
        scratch_shapes=[
            pltpu.VMEM((_NS, _D), jnp.float32),
            pltpu.VMEM((2, _NS, _D), jnp.bfloat16),
        ],
    )
    out = pl.pallas_call(
        _hops_body,
        grid_spec=hops,
        out_shape=jax.ShapeDtypeStruct((_N, _D), jnp.float32),
        compiler_params=pltpu.CompilerParams(
            dimension_semantics=("arbitrary", "arbitrary"),
        ),
    )(temp, h1, acc1, a_b)

    return out
